# baseline (speedup 1.0000x reference)
import functools
import math

import jax
import jax.numpy as jnp
from jax import lax
from jax.experimental import pallas as pl
from jax.experimental.pallas import tpu as pltpu
from jax.experimental.pallas import tpu_sc as plsc

D_LANES = 16

NUM_CORES = 2
NUM_SUBCORES = 16
NUM_WORKERS = NUM_CORES * NUM_SUBCORES


@functools.lru_cache(maxsize=None)
def _make_embed(B, V, D, chunk):
    assert B % (NUM_WORKERS * chunk) == 0 and chunk % 2 == 0
    b_per_w = B // NUM_WORKERS
    n = b_per_w // chunk
    assert n % 2 == 0 and n >= 4
    scale = math.sqrt(D)
    vregs = D // D_LANES

    mesh = plsc.VectorSubcoreMesh(
        core_axis_name="c", subcore_axis_name="s")

    @functools.partial(
        pl.kernel,
        out_type=jax.ShapeDtypeStruct((B // 2, 2 * D), jnp.float32),
        mesh=mesh,
        scratch_types=[
            pltpu.VMEM((b_per_w,), jnp.int32),
            pltpu.VMEM((chunk,), jnp.int32),
            pltpu.VMEM((chunk,), jnp.int32),
            pltpu.VMEM((chunk, 2 * D), jnp.float32),
            pltpu.VMEM((chunk, 2 * D), jnp.float32),
            pltpu.VMEM((chunk // 2, 2 * D), jnp.float32),
            pltpu.VMEM((chunk // 2, 2 * D), jnp.float32),
            pltpu.SemaphoreType.DMA,
            pltpu.SemaphoreType.DMA,
            pltpu.SemaphoreType.DMA,
            pltpu.SemaphoreType.DMA,
        ],
    )
    def embed(table_hbm, idx_hbm, out_hbm, idx_v, p0, p1, g0, g1, o0, o1,
              gs0, gs1, ss0, ss1):
        wid = lax.axis_index("s") * NUM_CORES + lax.axis_index("c")
        base = pl.multiple_of(wid * b_per_w, 256)
        p_bufs = (p0, p1)
        g_bufs = (g0, g1)
        o_bufs = (o0, o1)
        g_sems = (gs0, gs1)
        s_sems = (ss0, ss1)

        pltpu.sync_copy(idx_hbm.at[pl.ds(base, b_per_w)], idx_v)

        def start_gather(c, p):
            @plsc.parallel_loop(0, chunk, D_LANES, unroll=4)
            def _(i):
                p_bufs[p][pl.ds(i, D_LANES)] = (
                    idx_v[pl.ds(c * chunk + i, D_LANES)] >> 1)

            pltpu.async_copy(table_hbm.at[p_bufs[p]], g_bufs[p], g_sems[p])

        def wait_gather(p):
            pltpu.make_async_copy(table_hbm.at[p_bufs[p]],
                                  g_bufs[p], g_sems[p]).wait()

        def scale_chunk(c, p):
            src = g_bufs[p]
            dst = o_bufs[p]
            cb = c * chunk

            @plsc.parallel_loop(0, chunk // D_LANES, unroll=1)
            def _(gi):
                r0 = gi * D_LANES
                qv = (idx_v[pl.ds(cb + r0, D_LANES)] & 1) * D
                for k in range(D_LANES // 2):
                    r = r0 + 2 * k
                    rp = gi * (D_LANES // 2) + k
                    off0 = qv[2 * k]
                    off1 = qv[2 * k + 1]
                    for j in range(vregs):
                        jo = j * D_LANES
                        dst[rp, pl.ds(jo, D_LANES)] = (
                            src[r, pl.ds(off0 + jo, D_LANES)] * scale)
                        dst[rp, pl.ds(D + jo, D_LANES)] = (
                            src[r + 1, pl.ds(off1 + jo, D_LANES)] * scale)

        def start_scatter(c, p):
            off = pl.multiple_of((base + c * chunk) // 2, 128)
            pltpu.async_copy(
                o_bufs[p], out_hbm.at[pl.ds(off, chunk // 2)], s_sems[p])

        def wait_scatter(p):
            pltpu.make_async_copy(
                o_bufs[p], out_hbm.at[pl.ds(0, chunk // 2)],
                s_sems[p]).wait()

        start_gather(0, 0)
        start_gather(1, 1)
        wait_gather(0)
        scale_chunk(0, 0)
        start_scatter(0, 0)
        start_gather(2, 0)
        wait_gather(1)
        scale_chunk(1, 1)
        start_scatter(1, 1)
        start_gather(3, 1)

        def pair_body(t, carry):
            c = 2 * t
            for p in range(2):
                wait_gather(p)
                wait_scatter(p)
                scale_chunk(c + p, p)
                start_scatter(c + p, p)
                start_gather(c + p + 2, p)
            return carry

        lax.fori_loop(1, n // 2 - 1, pair_body, 0)

        for p in range(2):
            wait_gather(p)
            wait_scatter(p)
            scale_chunk(n - 2 + p, p)
            start_scatter(n - 2 + p, p)
        wait_scatter(0)
        wait_scatter(1)

    return embed


def kernel(x, table):
    V, D = table.shape
    B = x.size
    tp = table.reshape(V // 2, 2 * D)
    out = _make_embed(B, V, D, 256)(tp, x.reshape(B))
    return out.reshape(*x.shape, D)


# speedup vs baseline: 1.0003x; 1.0003x over previous
import functools
import math

import jax
import jax.numpy as jnp
from jax import lax
from jax.experimental import pallas as pl
from jax.experimental.pallas import tpu as pltpu
from jax.experimental.pallas import tpu_sc as plsc

D_LANES = 16

NUM_CORES = 2
NUM_SUBCORES = 16
NUM_WORKERS = NUM_CORES * NUM_SUBCORES


@functools.lru_cache(maxsize=None)
def _make_embed(B, V, D, chunk):
    assert B % (NUM_WORKERS * chunk) == 0 and chunk % 2 == 0
    b_per_w = B // NUM_WORKERS
    n = b_per_w // chunk
    assert n % 2 == 0 and n >= 4
    scale = math.sqrt(D)
    vregs = D // D_LANES

    mesh = plsc.VectorSubcoreMesh(
        core_axis_name="c", subcore_axis_name="s")

    @functools.partial(
        pl.kernel,
        out_type=jax.ShapeDtypeStruct((B // 2, 2 * D), jnp.float32),
        mesh=mesh,
        scratch_types=[
            pltpu.VMEM((b_per_w,), jnp.int32),
            pltpu.VMEM((chunk,), jnp.int32),
            pltpu.VMEM((chunk,), jnp.int32),
            pltpu.VMEM((chunk, 2 * D), jnp.float32),
            pltpu.VMEM((chunk, 2 * D), jnp.float32),
            pltpu.VMEM((chunk // 2, 2 * D), jnp.float32),
            pltpu.VMEM((chunk // 2, 2 * D), jnp.float32),
            pltpu.SemaphoreType.DMA,
            pltpu.SemaphoreType.DMA,
            pltpu.SemaphoreType.DMA,
            pltpu.SemaphoreType.DMA,
        ],
        compiler_params=pltpu.CompilerParams(
            skip_device_barrier=True,
            disable_bounds_checks=True,
            disable_semaphore_checks=True,
        ),
    )
    def embed(table_hbm, idx_hbm, out_hbm, idx_v, p0, p1, g0, g1, o0, o1,
              gs0, gs1, ss0, ss1):
        wid = lax.axis_index("s") * NUM_CORES + lax.axis_index("c")
        base = pl.multiple_of(wid * b_per_w, 256)
        p_bufs = (p0, p1)
        g_bufs = (g0, g1)
        o_bufs = (o0, o1)
        g_sems = (gs0, gs1)
        s_sems = (ss0, ss1)

        pltpu.sync_copy(idx_hbm.at[pl.ds(base, b_per_w)], idx_v)

        def start_gather(c, p):
            @plsc.parallel_loop(0, chunk, D_LANES, unroll=4)
            def _(i):
                p_bufs[p][pl.ds(i, D_LANES)] = (
                    idx_v[pl.ds(c * chunk + i, D_LANES)] >> 1)

            pltpu.async_copy(table_hbm.at[p_bufs[p]], g_bufs[p], g_sems[p])

        def wait_gather(p):
            pltpu.make_async_copy(table_hbm.at[p_bufs[p]],
                                  g_bufs[p], g_sems[p]).wait()

        def scale_chunk(c, p):
            src = g_bufs[p]
            dst = o_bufs[p]
            cb = c * chunk

            @plsc.parallel_loop(0, chunk // D_LANES, unroll=1)
            def _(gi):
                r0 = gi * D_LANES
                qv = (idx_v[pl.ds(cb + r0, D_LANES)] & 1) * D
                for k in range(D_LANES // 2):
                    r = r0 + 2 * k
                    rp = gi * (D_LANES // 2) + k
                    off0 = qv[2 * k]
                    off1 = qv[2 * k + 1]
                    for j in range(vregs):
                        jo = j * D_LANES
                        dst[rp, pl.ds(jo, D_LANES)] = (
                            src[r, pl.ds(off0 + jo, D_LANES)] * scale)
                        dst[rp, pl.ds(D + jo, D_LANES)] = (
                            src[r + 1, pl.ds(off1 + jo, D_LANES)] * scale)

        def start_scatter(c, p):
            off = pl.multiple_of((base + c * chunk) // 2, 128)
            pltpu.async_copy(
                o_bufs[p], out_hbm.at[pl.ds(off, chunk // 2)], s_sems[p])

        def wait_scatter(p):
            pltpu.make_async_copy(
                o_bufs[p], out_hbm.at[pl.ds(0, chunk // 2)],
                s_sems[p]).wait()

        start_gather(0, 0)
        start_gather(1, 1)
        wait_gather(0)
        scale_chunk(0, 0)
        start_scatter(0, 0)
        start_gather(2, 0)
        wait_gather(1)
        scale_chunk(1, 1)
        start_scatter(1, 1)
        start_gather(3, 1)

        def pair_body(t, carry):
            c = 2 * t
            for p in range(2):
                wait_gather(p)
                wait_scatter(p)
                scale_chunk(c + p, p)
                start_scatter(c + p, p)
                start_gather(c + p + 2, p)
            return carry

        lax.fori_loop(1, n // 2 - 1, pair_body, 0)

        for p in range(2):
            wait_gather(p)
            wait_scatter(p)
            scale_chunk(n - 2 + p, p)
            start_scatter(n - 2 + p, p)
        wait_scatter(0)
        wait_scatter(1)

    return embed


def kernel(x, table):
    V, D = table.shape
    B = x.size
    tp = table.reshape(V // 2, 2 * D)
    out = _make_embed(B, V, D, 256)(tp, x.reshape(B))
    return out.reshape(*x.shape, D)


# speedup vs baseline: 1.0766x; 1.0763x over previous
import functools
import math

import jax
import jax.numpy as jnp
from jax import lax
from jax.experimental import pallas as pl
from jax.experimental.pallas import tpu as pltpu
from jax.experimental.pallas import tpu_sc as plsc

D_LANES = 16

NUM_CORES = 2
NUM_SUBCORES = 16
NUM_WORKERS = NUM_CORES * NUM_SUBCORES


@functools.lru_cache(maxsize=None)
def _make_embed(B, T, V, D):
    assert B % NUM_WORKERS == 0
    nb = B // NUM_WORKERS
    assert nb % 2 == 0 and nb >= 4
    scale = math.sqrt(D)
    vregs = D // D_LANES

    mesh = plsc.VectorSubcoreMesh(
        core_axis_name="c", subcore_axis_name="s")

    @functools.partial(
        pl.kernel,
        out_type=jax.ShapeDtypeStruct((B, T, D), jnp.float32),
        mesh=mesh,
        scratch_types=[
            pltpu.VMEM((nb, T), jnp.int32),
            pltpu.VMEM((T, D), jnp.float32),
            pltpu.VMEM((T, D), jnp.float32),
            pltpu.VMEM((1, T, D), jnp.float32),
            pltpu.VMEM((1, T, D), jnp.float32),
            pltpu.SemaphoreType.DMA,
            pltpu.SemaphoreType.DMA,
            pltpu.SemaphoreType.DMA,
            pltpu.SemaphoreType.DMA,
        ],
        compiler_params=pltpu.CompilerParams(use_tc_tiling_on_sc=False),
    )
    def embed(table_hbm, x_hbm, out_hbm, idx_v, g0, g1, o0, o1,
              gs0, gs1, ss0, ss1):
        wid = lax.axis_index("s") * NUM_CORES + lax.axis_index("c")
        base = wid * nb
        g_bufs = (g0, g1)
        o_bufs = (o0, o1)
        g_sems = (gs0, gs1)
        s_sems = (ss0, ss1)

        pltpu.sync_copy(x_hbm.at[pl.ds(base, nb)], idx_v)

        def start_gather(c, p):
            pltpu.async_copy(table_hbm.at[idx_v.at[c]], g_bufs[p], g_sems[p])

        def wait_gather(p):
            pltpu.make_async_copy(table_hbm.at[idx_v.at[0]],
                                  g_bufs[p], g_sems[p]).wait()

        def scale_chunk(p):
            src = g_bufs[p]
            dst = o_bufs[p]

            @plsc.parallel_loop(0, T, unroll=4)
            def _(r):
                for j in range(vregs):
                    sl = pl.ds(j * D_LANES, D_LANES)
                    dst[0, r, sl] = src[r, sl] * scale

        def start_scatter(c, p):
            pltpu.async_copy(
                o_bufs[p], out_hbm.at[pl.ds(base + c, 1)], s_sems[p])

        def wait_scatter(p):
            pltpu.make_async_copy(
                o_bufs[p], out_hbm.at[pl.ds(0, 1)], s_sems[p]).wait()

        start_gather(0, 0)
        start_gather(1, 1)
        wait_gather(0)
        scale_chunk(0)
        start_scatter(0, 0)
        start_gather(2, 0)
        wait_gather(1)
        scale_chunk(1)
        start_scatter(1, 1)
        start_gather(3, 1)

        def pair_body(t, carry):
            c = 2 * t
            for p in range(2):
                wait_gather(p)
                wait_scatter(p)
                scale_chunk(p)
                start_scatter(c + p, p)
                start_gather(c + p + 2, p)
            return carry

        lax.fori_loop(1, nb // 2 - 1, pair_body, 0)

        for p in range(2):
            wait_gather(p)
            wait_scatter(p)
            scale_chunk(p)
            start_scatter(nb - 2 + p, p)
        wait_scatter(0)
        wait_scatter(1)

    return embed


def kernel(x, table):
    V, D = table.shape
    B, T = x.shape
    return _make_embed(B, T, V, D)(table, x)


# speedup vs baseline: 1.1405x; 1.0593x over previous
import functools
import math

import jax
import jax.numpy as jnp
from jax import lax
from jax.experimental import pallas as pl
from jax.experimental.pallas import tpu as pltpu
from jax.experimental.pallas import tpu_sc as plsc

D_LANES = 16

NUM_CORES = 2
NUM_SUBCORES = 16
NUM_WORKERS = NUM_CORES * NUM_SUBCORES

CHUNK = 128


@functools.lru_cache(maxsize=None)
def _make_embed(N, V2, D):
    assert N % (NUM_WORKERS * CHUNK) == 0
    nchunk = N // (NUM_WORKERS * CHUNK)
    assert nchunk % 2 == 0 and nchunk >= 6
    vregs = D // D_LANES
    ngrp = CHUNK // D_LANES
    scale = math.sqrt(D)

    mesh = plsc.VectorSubcoreMesh(core_axis_name="c", subcore_axis_name="s")

    @functools.partial(
        pl.kernel,
        out_type=jax.ShapeDtypeStruct((N, D), jnp.float32),
        mesh=mesh,
        scratch_types=[
            pltpu.VMEM((CHUNK,), jnp.int32),
            pltpu.VMEM((CHUNK,), jnp.int32),
            pltpu.VMEM((CHUNK,), jnp.int32),
            pltpu.VMEM((CHUNK,), jnp.int32),
            pltpu.VMEM((CHUNK,), jnp.int32),
            pltpu.VMEM((CHUNK,), jnp.int32),
            pltpu.VMEM((CHUNK, 2 * D), jnp.float32),
            pltpu.VMEM((CHUNK, 2 * D), jnp.float32),
            pltpu.VMEM((CHUNK, D), jnp.float32),
            pltpu.VMEM((CHUNK, D), jnp.float32),
            pltpu.SemaphoreType.DMA,
            pltpu.SemaphoreType.DMA,
            pltpu.SemaphoreType.DMA,
            pltpu.SemaphoreType.DMA,
            pltpu.SemaphoreType.DMA,
            pltpu.SemaphoreType.DMA,
        ],
    )
    def embed(pair_hbm, x_hbm, out_hbm, i0, i1, p0, p1, q0, q1,
              g0, g1, o0, o1, is0, is1, gs0, gs1, ss0, ss1):
        wid = lax.axis_index("s") * NUM_CORES + lax.axis_index("c")
        base = pl.multiple_of(wid * (nchunk * CHUNK), CHUNK)
        i_bufs = (i0, i1)
        p_bufs = (p0, p1)
        q_bufs = (q0, q1)
        g_bufs = (g0, g1)
        o_bufs = (o0, o1)
        i_sems = (is0, is1)
        g_sems = (gs0, gs1)
        s_sems = (ss0, ss1)

        def start_idx(c, p):
            pltpu.async_copy(
                x_hbm.at[pl.ds(base + c * CHUNK, CHUNK)], i_bufs[p], i_sems[p])

        def wait_idx(p):
            pltpu.make_async_copy(
                x_hbm.at[pl.ds(0, CHUNK)], i_bufs[p], i_sems[p]).wait()

        def compute_ids(p):
            ib, pb, qb = i_bufs[p], p_bufs[p], q_bufs[p]
            for k in range(ngrp):
                sl = pl.ds(k * D_LANES, D_LANES)
                iv = ib[sl]
                pb[sl] = iv >> 1
                qb[sl] = (iv & 1) * D

        def start_gather(p):
            pltpu.async_copy(pair_hbm.at[p_bufs[p]], g_bufs[p], g_sems[p])

        def wait_gather(p):
            pltpu.make_async_copy(
                pair_hbm.at[p_bufs[p]], g_bufs[p], g_sems[p]).wait()

        def select_scale(p):
            gb, ob, qb = g_bufs[p], o_bufs[p], q_bufs[p]

            @plsc.parallel_loop(0, ngrp, unroll=2)
            def _(gi):
                r0 = gi * D_LANES
                offv = qb[pl.ds(r0, D_LANES)]
                for m in range(D_LANES):
                    off = offv[m]
                    for j in range(vregs):
                        src = pl.multiple_of(off + j * D_LANES, D_LANES)
                        ob[r0 + m, pl.ds(j * D_LANES, D_LANES)] = (
                            gb[r0 + m, pl.ds(src, D_LANES)] * scale)

        def start_scatter(c, p):
            pltpu.async_copy(
                o_bufs[p], out_hbm.at[pl.ds(base + c * CHUNK, CHUNK)], s_sems[p])

        def wait_scatter(p):
            pltpu.make_async_copy(
                o_bufs[p], out_hbm.at[pl.ds(0, CHUNK)], s_sems[p]).wait()

        def chunk_step(c, p, first=False, last=False, no_more_idx=False):
            if not last:
                wait_idx(1 - p)
                compute_ids(1 - p)
                if not first:
                    wait_scatter(1 - p)
                start_gather(1 - p)
            wait_gather(p)
            if not (last or no_more_idx):
                start_idx(c + 2, p)
            select_scale(p)
            start_scatter(c, p)

        start_idx(0, 0)
        start_idx(1, 1)
        wait_idx(0)
        compute_ids(0)
        start_gather(0)

        chunk_step(0, 0, first=True)
        chunk_step(1, 1)

        def pair_body(t, carry):
            chunk_step(2 * t, 0)
            chunk_step(2 * t + 1, 1)
            return carry

        lax.fori_loop(1, nchunk // 2 - 1, pair_body, 0)

        chunk_step(nchunk - 2, 0, no_more_idx=True)
        chunk_step(nchunk - 1, 1, last=True)

        wait_scatter(0)
        wait_scatter(1)

    return embed


def kernel(x, table):
    V, D = table.shape
    B, T = x.shape
    pair = table.reshape(V // 2, 2 * D)
    out = _make_embed(B * T, V // 2, D)(pair, x.reshape(B * T))
    return out.reshape(B, T, D)


# speedup vs baseline: 1.1466x; 1.0054x over previous
import functools
import math

import jax
import jax.numpy as jnp
from jax import lax
from jax.experimental import pallas as pl
from jax.experimental.pallas import tpu as pltpu
from jax.experimental.pallas import tpu_sc as plsc

D_LANES = 16

NUM_CORES = 2
NUM_SUBCORES = 16
NUM_WORKERS = NUM_CORES * NUM_SUBCORES

CHUNK = 128


@functools.lru_cache(maxsize=None)
def _make_pack(V, D):
    V2 = V // 2
    blk = 5000
    assert V2 % blk == 0
    nb = V2 // blk

    def body(a_ref, b_ref, o_ref):
        o_ref[...] = jnp.concatenate([a_ref[...], b_ref[...]], axis=1)

    return pl.pallas_call(
        body,
        grid=(nb,),
        in_specs=[
            pl.BlockSpec((blk, D), lambda i: (i, 0)),
            pl.BlockSpec((blk, D), lambda i: (i + nb, 0)),
        ],
        out_specs=pl.BlockSpec((blk, 2 * D), lambda i: (i, 0)),
        out_shape=jax.ShapeDtypeStruct((V2, 2 * D), jnp.float32),
    )


@functools.lru_cache(maxsize=None)
def _make_embed(N, V2, D):
    assert N % (NUM_WORKERS * CHUNK) == 0
    nchunk = N // (NUM_WORKERS * CHUNK)
    assert nchunk % 2 == 0 and nchunk >= 6
    vregs = D // D_LANES
    ngrp = CHUNK // D_LANES
    scale = math.sqrt(D)

    mesh = plsc.VectorSubcoreMesh(core_axis_name="c", subcore_axis_name="s")

    @functools.partial(
        pl.kernel,
        out_type=jax.ShapeDtypeStruct((N, D), jnp.float32),
        mesh=mesh,
        scratch_types=[
            pltpu.VMEM((CHUNK,), jnp.int32),
            pltpu.VMEM((CHUNK,), jnp.int32),
            pltpu.VMEM((CHUNK,), jnp.int32),
            pltpu.VMEM((CHUNK,), jnp.int32),
            pltpu.VMEM((CHUNK,), jnp.int32),
            pltpu.VMEM((CHUNK,), jnp.int32),
            pltpu.VMEM((CHUNK, 2 * D), jnp.float32),
            pltpu.VMEM((CHUNK, 2 * D), jnp.float32),
            pltpu.VMEM((CHUNK, D), jnp.float32),
            pltpu.VMEM((CHUNK, D), jnp.float32),
            pltpu.SemaphoreType.DMA,
            pltpu.SemaphoreType.DMA,
            pltpu.SemaphoreType.DMA,
            pltpu.SemaphoreType.DMA,
            pltpu.SemaphoreType.DMA,
            pltpu.SemaphoreType.DMA,
        ],
    )
    def embed(pair_hbm, x_hbm, out_hbm, i0, i1, p0, p1, q0, q1,
              g0, g1, o0, o1, is0, is1, gs0, gs1, ss0, ss1):
        wid = lax.axis_index("s") * NUM_CORES + lax.axis_index("c")
        base = pl.multiple_of(wid * (nchunk * CHUNK), CHUNK)
        i_bufs = (i0, i1)
        p_bufs = (p0, p1)
        q_bufs = (q0, q1)
        g_bufs = (g0, g1)
        o_bufs = (o0, o1)
        i_sems = (is0, is1)
        g_sems = (gs0, gs1)
        s_sems = (ss0, ss1)

        def start_idx(c, p):
            pltpu.async_copy(
                x_hbm.at[pl.ds(base + c * CHUNK, CHUNK)], i_bufs[p], i_sems[p])

        def wait_idx(p):
            pltpu.make_async_copy(
                x_hbm.at[pl.ds(0, CHUNK)], i_bufs[p], i_sems[p]).wait()

        def compute_ids(p):
            ib, pb, qb = i_bufs[p], p_bufs[p], q_bufs[p]
            for k in range(ngrp):
                sl = pl.ds(k * D_LANES, D_LANES)
                iv = ib[sl]
                big = iv >= V2
                pb[sl] = iv - jnp.where(big, V2, 0)
                qb[sl] = jnp.where(big, D, 0)

        def start_gather(p):
            pltpu.async_copy(pair_hbm.at[p_bufs[p]], g_bufs[p], g_sems[p])

        def wait_gather(p):
            pltpu.make_async_copy(
                pair_hbm.at[p_bufs[p]], g_bufs[p], g_sems[p]).wait()

        def select_scale(p):
            gb, ob, qb = g_bufs[p], o_bufs[p], q_bufs[p]

            @plsc.parallel_loop(0, ngrp, unroll=2)
            def _(gi):
                r0 = gi * D_LANES
                offv = qb[pl.ds(r0, D_LANES)]
                for m in range(D_LANES):
                    off = offv[m]
                    for j in range(vregs):
                        src = pl.multiple_of(off + j * D_LANES, D_LANES)
                        ob[r0 + m, pl.ds(j * D_LANES, D_LANES)] = (
                            gb[r0 + m, pl.ds(src, D_LANES)] * scale)

        def start_scatter(c, p):
            pltpu.async_copy(
                o_bufs[p], out_hbm.at[pl.ds(base + c * CHUNK, CHUNK)], s_sems[p])

        def wait_scatter(p):
            pltpu.make_async_copy(
                o_bufs[p], out_hbm.at[pl.ds(0, CHUNK)], s_sems[p]).wait()

        def chunk_step(c, p, first=False, last=False, no_more_idx=False):
            if not last:
                wait_idx(1 - p)
                compute_ids(1 - p)
                if not first:
                    wait_scatter(1 - p)
                start_gather(1 - p)
            wait_gather(p)
            if not (last or no_more_idx):
                start_idx(c + 2, p)
            select_scale(p)
            start_scatter(c, p)

        start_idx(0, 0)
        start_idx(1, 1)
        wait_idx(0)
        compute_ids(0)
        start_gather(0)

        chunk_step(0, 0, first=True)
        chunk_step(1, 1)

        def pair_body(t, carry):
            chunk_step(2 * t, 0)
            chunk_step(2 * t + 1, 1)
            return carry

        lax.fori_loop(1, nchunk // 2 - 1, pair_body, 0)

        chunk_step(nchunk - 2, 0, no_more_idx=True)
        chunk_step(nchunk - 1, 1, last=True)

        wait_scatter(0)
        wait_scatter(1)

    return embed


def kernel(x, table):
    V, D = table.shape
    B, T = x.shape
    pair = _make_pack(V, D)(table, table)
    out = _make_embed(B * T, V // 2, D)(pair, x.reshape(B * T))
    return out.reshape(B, T, D)
